# baseline (speedup 1.0000x reference)
import functools

import jax
import jax.numpy as jnp
from jax import lax
from jax.experimental import pallas as pl
from jax.experimental.pallas import tpu as pltpu
from jax.experimental.pallas import tpu_sc as plsc

NUM_TYPES = 10000
D = 128
B = 16
L = 200
LP = 256
NW = 32
NSPLIT = 1

AMAIN = 9984


def _sc_gather_g(ev_flat, a, a_tail, nb):
    nwpb = NW // nb
    rpw = L // nwpb
    kr = rpw // 25
    gpr = 16 // kr
    ng = rpw // kr
    nt = ((rpw + 15) // 16) * 16
    mesh = plsc.VectorSubcoreMesh(core_axis_name="c", subcore_axis_name="s")

    @functools.partial(
        pl.kernel,
        mesh=mesh,
        out_type=jax.ShapeDtypeStruct((NW, rpw, LP), jnp.float32),
        scratch_types=[
            pltpu.VMEM((LP,), jnp.int32),
            pltpu.VMEM((nt,), jnp.int32),
            pltpu.VMEM((nt // 16, 16), jnp.int32),
            pltpu.VMEM((nt, 128), jnp.float32),
            pltpu.VMEM((kr // 2, AMAIN), jnp.float32),
            pltpu.VMEM((kr // 2, AMAIN), jnp.float32),
            pltpu.VMEM((kr // 2, AMAIN), jnp.float32),
            pltpu.VMEM((kr // 2, AMAIN), jnp.float32),
            pltpu.VMEM((rpw, LP), jnp.float32),
            pltpu.SemaphoreType.DMA,
            pltpu.SemaphoreType.DMA,
            pltpu.SemaphoreType.DMA,
        ],
        compiler_params=pltpu.CompilerParams(needs_layout_passes=False),
    )
    def k(ev_hbm, a_hbm, atail_hbm, out_hbm,
          ev_v, rids_v, rids2_v, tail_v, buf0a, buf0b, buf1a, buf1b, rows_v,
          sem_t, s0, s1):
        bufs = ((buf0a, buf0b), (buf1a, buf1b))
        sems = (s0, s1)
        wid = lax.axis_index("s") * 2 + lax.axis_index("c")
        b = wid // nwpb
        i0 = (wid % nwpb) * rpw

        zeros16 = jnp.zeros((16,), jnp.int32)
        for cc in range(L // 16, LP // 16):
            ev_v[pl.ds(cc * 16, 16)] = zeros16
        pltpu.sync_copy(ev_hbm.at[pl.ds(b * L, L)], ev_v.at[pl.ds(0, L)])

        for cc in range(nt // 16):
            ids = ev_v[pl.ds(i0 + cc * 16, 16)]
            rids_v[pl.ds(cc * 16, 16)] = ids
            rids2_v[cc, pl.ds(0, 16)] = ids
        cp_tail = pltpu.async_copy(atail_hbm.at[rids_v], tail_v, sem_t)

        kh = kr // 2

        def fire(g, t):
            for h in range(2):
                idx = rids2_v.at[g // gpr, pl.ds((g % gpr) * kr + h * kh, kh)]
                pltpu.async_copy(a_hbm.at[idx, pl.ds(0, AMAIN)],
                                 bufs[t][h], sems[t])

        def wait_buf(t):
            for h in range(2):
                pltpu.make_async_copy(
                    a_hbm.at[pl.ds(0, kh), pl.ds(0, AMAIN)],
                    bufs[t][h], sems[t]).wait()

        fire(0, 0)
        cp_tail.wait()

        def extract_group(g, bufpair):
            for rr in range(kr):
                r = g * kr + rr
                frr = jnp.full((16,), rr % kh, jnp.int32)
                fr = jnp.full((16,), r, jnp.int32)
                for cc in range(LP // 16):
                    ev_c = ev_v[pl.ds(cc * 16, 16)]
                    vm = plsc.load_gather(
                        bufpair[rr // kh], [frr, jnp.minimum(ev_c, AMAIN - 1)])
                    vt = plsc.load_gather(
                        tail_v, [fr, jnp.maximum(ev_c - AMAIN, 0)])
                    rows_v[r, pl.ds(cc * 16, 16)] = jnp.where(
                        ev_c < AMAIN, vm, vt)

        def step(q, carry):
            for par in range(2):
                g = q * 2 + par
                fire(g + 1, 1 - par)
                wait_buf(par)
                extract_group(g, bufs[par])
            return carry

        lax.fori_loop(0, ng // 2, step, 0)
        wait_buf(0)
        extract_group(ng - 1, bufs[0])

        pltpu.sync_copy(rows_v, out_hbm.at[wid])

    return k(ev_flat, a, a_tail)


def _tc_encoder(gp, x, w0, b0, w1, b1, nb):
    nwpb = NW // nb
    rpw = L // nwpb

    def body(*refs):
        gp_refs = refs[:nwpb]
        x_ref, w0_ref, b0_ref, w1_ref, b1_ref, out_ref = refs[nwpb:]
        gp = jnp.concatenate([gr[0] for gr in gp_refs], axis=0)
        ri = lax.broadcasted_iota(jnp.int32, (L, LP), 0)
        ci = lax.broadcasted_iota(jnp.int32, (L, LP), 1)
        diag = jnp.sum(jnp.where(ri == ci, gp, 0.0), axis=0, keepdims=True)
        adj = gp + diag
        xx = x_ref[0]
        zpad = jnp.zeros((LP - L, D), jnp.float32)
        for w_ref, bias_ref in ((w0_ref, b0_ref), (w1_ref, b1_ref)):
            xp = jnp.concatenate([xx, zpad], axis=0)
            h = jnp.dot(adj, xp, preferred_element_type=jnp.float32)
            hw = jnp.dot(h, w_ref[...], preferred_element_type=jnp.float32)
            xx = jnp.maximum(hw + bias_ref[...], 0.0) + xx
        out_ref[...] = jnp.mean(xx, axis=0, keepdims=True)[None]

    gp_specs = [
        pl.BlockSpec((1, rpw, LP), lambda i, s=s: (nwpb * i + s, 0, 0))
        for s in range(nwpb)
    ]
    return pl.pallas_call(
        body,
        grid=(nb,),
        in_specs=gp_specs + [
            pl.BlockSpec((1, L, D), lambda i: (i, 0, 0)),
            pl.BlockSpec((D, D), lambda i: (0, 0)),
            pl.BlockSpec((1, D), lambda i: (0, 0)),
            pl.BlockSpec((D, D), lambda i: (0, 0)),
            pl.BlockSpec((1, D), lambda i: (0, 0)),
        ],
        out_specs=pl.BlockSpec((1, 1, D), lambda i: (i, 0, 0)),
        out_shape=jax.ShapeDtypeStruct((nb, 1, D), jnp.float32),
    )(*([gp] * nwpb), x, w0, b0, w1, b1).reshape(nb, D)


@jax.jit
def kernel(event_type, enc_output, slf_attn_mask, non_pad_mask, ui_adj,
           W0, b0, W1, b1):
    ev = (event_type - 1).astype(jnp.int32).reshape(B * L)
    a_tail = jnp.pad(ui_adj[:, AMAIN:], ((0, 0), (0, 128 - (NUM_TYPES - AMAIN))))
    eye = jnp.eye(W0.shape[0], dtype=jnp.float32)
    wbd0 = (eye[:, None, :, None] * W0[:, :, None, :]).reshape(D, D)
    wbd1 = (eye[:, None, :, None] * W1[:, :, None, :]).reshape(D, D)
    b0r = b0.reshape(1, D)
    b1r = b1.reshape(1, D)

    nb = B // NSPLIT
    outs = []
    for s in range(NSPLIT):
        gp = _sc_gather_g(ev[s * nb * L:(s + 1) * nb * L], ui_adj, a_tail, nb)
        outs.append(_tc_encoder(
            gp, enc_output[s * nb:(s + 1) * nb], wbd0, b0r, wbd1, b1r, nb))
    return jnp.concatenate(outs, axis=0)


# speedup vs baseline: 1.0401x; 1.0401x over previous
import functools

import jax
import jax.numpy as jnp
from jax import lax
from jax.experimental import pallas as pl
from jax.experimental.pallas import tpu as pltpu
from jax.experimental.pallas import tpu_sc as plsc

NUM_TYPES = 10000
D = 128
B = 16
L = 200
LP = 256
NW = 32
NSPLIT = 1

AMAIN = 9984


def _sc_gather_g(ev_flat, a, a_tail, nb):
    nwpb = NW // nb
    rpw = L // nwpb
    kr = rpw // 25
    gpr = 16 // kr
    ng = rpw // kr
    nt = ((rpw + 15) // 16) * 16
    mesh = plsc.VectorSubcoreMesh(core_axis_name="c", subcore_axis_name="s")

    @functools.partial(
        pl.kernel,
        mesh=mesh,
        out_type=jax.ShapeDtypeStruct((NW, rpw, LP), jnp.float32),
        scratch_types=[
            pltpu.VMEM((LP,), jnp.int32),
            pltpu.VMEM((nt,), jnp.int32),
            pltpu.VMEM((nt // 16, 16), jnp.int32),
            pltpu.VMEM((nt, 128), jnp.float32),
            pltpu.VMEM((kr // 2, AMAIN), jnp.float32),
            pltpu.VMEM((kr // 2, AMAIN), jnp.float32),
            pltpu.VMEM((kr // 2, AMAIN), jnp.float32),
            pltpu.VMEM((kr // 2, AMAIN), jnp.float32),
            pltpu.VMEM((rpw, LP), jnp.float32),
            pltpu.SemaphoreType.DMA,
            pltpu.SemaphoreType.DMA,
            pltpu.SemaphoreType.DMA,
        ],
        compiler_params=pltpu.CompilerParams(needs_layout_passes=False),
    )
    def k(ev_hbm, a_hbm, atail_hbm, out_hbm,
          ev_v, rids_v, rids2_v, tail_v, buf0a, buf0b, buf1a, buf1b, rows_v,
          sem_t, s0, s1):
        bufs = ((buf0a, buf0b), (buf1a, buf1b))
        sems = (s0, s1)
        wid = lax.axis_index("s") * 2 + lax.axis_index("c")
        b = wid // nwpb
        i0 = (wid % nwpb) * rpw

        zeros16 = jnp.zeros((16,), jnp.int32)
        for cc in range(L // 16, LP // 16):
            ev_v[pl.ds(cc * 16, 16)] = zeros16
        pltpu.sync_copy(ev_hbm.at[pl.ds(b * L, L)], ev_v.at[pl.ds(0, L)])

        for cc in range(nt // 16):
            ids = ev_v[pl.ds(i0 + cc * 16, 16)]
            rids_v[pl.ds(cc * 16, 16)] = ids
            rids2_v[cc, pl.ds(0, 16)] = ids
        cp_tail = pltpu.async_copy(atail_hbm.at[rids_v], tail_v, sem_t)

        kh = kr // 2

        def fire(g, t):
            for h in range(2):
                idx = rids2_v.at[g // gpr, pl.ds((g % gpr) * kr + h * kh, kh)]
                pltpu.async_copy(a_hbm.at[idx, pl.ds(0, AMAIN)],
                                 bufs[t][h], sems[t])

        def wait_buf(t):
            for h in range(2):
                pltpu.make_async_copy(
                    a_hbm.at[pl.ds(0, kh), pl.ds(0, AMAIN)],
                    bufs[t][h], sems[t]).wait()

        fire(0, 0)
        cp_tail.wait()

        def extract_group(g, bufpair):
            for rr in range(kr):
                r = g * kr + rr
                frr = jnp.full((16,), rr % kh, jnp.int32)
                fr = jnp.full((16,), r, jnp.int32)
                for cc in range(LP // 16):
                    ev_c = ev_v[pl.ds(cc * 16, 16)]
                    vm = plsc.load_gather(
                        bufpair[rr // kh], [frr, jnp.minimum(ev_c, AMAIN - 1)])
                    vt = plsc.load_gather(
                        tail_v, [fr, jnp.maximum(ev_c - AMAIN, 0)])
                    rows_v[r, pl.ds(cc * 16, 16)] = jnp.where(
                        ev_c < AMAIN, vm, vt)

        def step(q, carry):
            for par in range(2):
                g = q * 2 + par
                fire(g + 1, 1 - par)
                wait_buf(par)
                extract_group(g, bufs[par])
            return carry

        lax.fori_loop(0, ng // 2, step, 0)
        wait_buf(0)
        extract_group(ng - 1, bufs[0])

        pltpu.sync_copy(rows_v, out_hbm.at[wid])

    return k(ev_flat, a, a_tail)


def _tc_encoder(gp, x, w0, b0, w1, b1, nb):
    nwpb = NW // nb
    rpw = L // nwpb
    bpg = 2
    nslab = nwpb * bpg

    def body(*refs):
        gp_refs = refs[:nslab]
        x_ref, w0_ref, b0_ref, w1_ref, b1_ref, out_ref = refs[nslab:]
        ri = lax.broadcasted_iota(jnp.int32, (L, LP), 0)
        ci = lax.broadcasted_iota(jnp.int32, (L, LP), 1)
        zpad = jnp.zeros((LP - L, D), jnp.float32)
        for bb in range(bpg):
            gpb = jnp.concatenate(
                [gr[0] for gr in gp_refs[bb * nwpb:(bb + 1) * nwpb]], axis=0)
            diag = jnp.sum(jnp.where(ri == ci, gpb, 0.0), axis=0,
                           keepdims=True)
            adj = gpb + diag
            xx = x_ref[bb]
            for w_ref, bias_ref in ((w0_ref, b0_ref), (w1_ref, b1_ref)):
                xp = jnp.concatenate([xx, zpad], axis=0)
                h = jnp.dot(adj, xp, preferred_element_type=jnp.float32)
                hw = jnp.dot(h, w_ref[...], preferred_element_type=jnp.float32)
                xx = jnp.maximum(hw + bias_ref[...], 0.0) + xx
            out_ref[bb, :, :] = jnp.mean(xx, axis=0, keepdims=True)

    gp_specs = [
        pl.BlockSpec((1, rpw, LP), lambda i, s=s: (nslab * i + s, 0, 0))
        for s in range(nslab)
    ]
    return pl.pallas_call(
        body,
        grid=(nb // bpg,),
        in_specs=gp_specs + [
            pl.BlockSpec((bpg, L, D), lambda i: (i, 0, 0)),
            pl.BlockSpec((D, D), lambda i: (0, 0)),
            pl.BlockSpec((1, D), lambda i: (0, 0)),
            pl.BlockSpec((D, D), lambda i: (0, 0)),
            pl.BlockSpec((1, D), lambda i: (0, 0)),
        ],
        out_specs=pl.BlockSpec((bpg, 1, D), lambda i: (i, 0, 0)),
        out_shape=jax.ShapeDtypeStruct((nb, 1, D), jnp.float32),
    )(*([gp] * nslab), x, w0, b0, w1, b1).reshape(nb, D)


@jax.jit
def kernel(event_type, enc_output, slf_attn_mask, non_pad_mask, ui_adj,
           W0, b0, W1, b1):
    ev = (event_type - 1).astype(jnp.int32).reshape(B * L)
    a_tail = jnp.pad(ui_adj[:, AMAIN:], ((0, 0), (0, 128 - (NUM_TYPES - AMAIN))))
    eye = jnp.eye(W0.shape[0], dtype=jnp.float32)
    wbd0 = (eye[:, None, :, None] * W0[:, :, None, :]).reshape(D, D)
    wbd1 = (eye[:, None, :, None] * W1[:, :, None, :]).reshape(D, D)
    b0r = b0.reshape(1, D)
    b1r = b1.reshape(1, D)

    nb = B // NSPLIT
    outs = []
    for s in range(NSPLIT):
        gp = _sc_gather_g(ev[s * nb * L:(s + 1) * nb * L], ui_adj, a_tail, nb)
        outs.append(_tc_encoder(
            gp, enc_output[s * nb:(s + 1) * nb], wbd0, b0r, wbd1, b1r, nb))
    return jnp.concatenate(outs, axis=0)


# speedup vs baseline: 1.0557x; 1.0150x over previous
import functools

import jax
import jax.numpy as jnp
from jax import lax
from jax.experimental import pallas as pl
from jax.experimental.pallas import tpu as pltpu
from jax.experimental.pallas import tpu_sc as plsc

NUM_TYPES = 10000
D = 128
B = 16
L = 200
LP = 256
NW = 32
NSPLIT = 1

AMAIN = 9984


def _sc_gather_g(ev_flat, a, a_tail, nb):
    nwpb = NW // nb
    rpw = L // nwpb
    kr = rpw // 25
    gpr = 16 // kr
    ng = rpw // kr
    nt = ((rpw + 15) // 16) * 16
    mesh = plsc.VectorSubcoreMesh(core_axis_name="c", subcore_axis_name="s")

    @functools.partial(
        pl.kernel,
        mesh=mesh,
        out_type=jax.ShapeDtypeStruct((NW, rpw, LP), jnp.float32),
        scratch_types=[
            pltpu.VMEM((LP,), jnp.int32),
            pltpu.VMEM((nt,), jnp.int32),
            pltpu.VMEM((nt // 16, 16), jnp.int32),
            pltpu.VMEM((nt, 128), jnp.float32),
            pltpu.VMEM((kr // 2, AMAIN), jnp.float32),
            pltpu.VMEM((kr // 2, AMAIN), jnp.float32),
            pltpu.VMEM((kr // 2, AMAIN), jnp.float32),
            pltpu.VMEM((kr // 2, AMAIN), jnp.float32),
            pltpu.VMEM((rpw, LP), jnp.float32),
            pltpu.SemaphoreType.DMA,
            pltpu.SemaphoreType.DMA,
            pltpu.SemaphoreType.DMA,
        ],
        compiler_params=pltpu.CompilerParams(needs_layout_passes=False),
    )
    def k(ev_hbm, a_hbm, atail_hbm, out_hbm,
          ev_v, rids_v, rids2_v, tail_v, buf0a, buf0b, buf1a, buf1b, rows_v,
          sem_t, s0, s1):
        bufs = ((buf0a, buf0b), (buf1a, buf1b))
        sems = (s0, s1)
        wid = lax.axis_index("s") * 2 + lax.axis_index("c")
        b = wid // nwpb
        i0 = (wid % nwpb) * rpw

        zeros16 = jnp.zeros((16,), jnp.int32)
        for cc in range(L // 16, LP // 16):
            ev_v[pl.ds(cc * 16, 16)] = zeros16
        pltpu.sync_copy(ev_hbm.at[pl.ds(b * L, L)], ev_v.at[pl.ds(0, L)])

        for cc in range(nt // 16):
            ids = ev_v[pl.ds(i0 + cc * 16, 16)]
            rids_v[pl.ds(cc * 16, 16)] = ids
            rids2_v[cc, pl.ds(0, 16)] = ids
        cp_tail = pltpu.async_copy(atail_hbm.at[rids_v], tail_v, sem_t)

        kh = kr // 2

        def fire(g, t):
            for h in range(2):
                idx = rids2_v.at[g // gpr, pl.ds((g % gpr) * kr + h * kh, kh)]
                pltpu.async_copy(a_hbm.at[idx, pl.ds(0, AMAIN)],
                                 bufs[t][h], sems[t])

        def wait_buf(t):
            for h in range(2):
                pltpu.make_async_copy(
                    a_hbm.at[pl.ds(0, kh), pl.ds(0, AMAIN)],
                    bufs[t][h], sems[t]).wait()

        fire(0, 0)
        cp_tail.wait()

        def extract_group(g, bufpair):
            for rr in range(kr):
                r = g * kr + rr
                frr = jnp.full((16,), rr % kh, jnp.int32)
                fr = jnp.full((16,), r, jnp.int32)
                for cc in range(LP // 16):
                    ev_c = ev_v[pl.ds(cc * 16, 16)]
                    vm = plsc.load_gather(
                        bufpair[rr // kh], [frr, jnp.minimum(ev_c, AMAIN - 1)])
                    vt = plsc.load_gather(
                        tail_v, [fr, jnp.maximum(ev_c - AMAIN, 0)])
                    rows_v[r, pl.ds(cc * 16, 16)] = jnp.where(
                        ev_c < AMAIN, vm, vt)

        def step(q, carry):
            for par in range(2):
                g = q * 2 + par
                fire(g + 1, 1 - par)
                wait_buf(par)
                extract_group(g, bufs[par])
            return carry

        lax.fori_loop(0, ng // 2, step, 0)
        wait_buf(0)
        extract_group(ng - 1, bufs[0])

        pltpu.sync_copy(rows_v, out_hbm.at[wid])

    return k(ev_flat, a, a_tail)


def _tc_encoder(gp, x, w0, b0, w1, b1, nb):
    nwpb = NW // nb
    rpw = L // nwpb
    bpg = 4
    nslab = nwpb * bpg

    def body(*refs):
        gp_refs = refs[:nslab]
        x_ref, w0_ref, b0_ref, w1_ref, b1_ref, out_ref = refs[nslab:]
        ri = lax.broadcasted_iota(jnp.int32, (L, LP), 0)
        ci = lax.broadcasted_iota(jnp.int32, (L, LP), 1)
        zpad = jnp.zeros((LP - L, D), jnp.float32)
        for bb in range(bpg):
            gpb = jnp.concatenate(
                [gr[0] for gr in gp_refs[bb * nwpb:(bb + 1) * nwpb]], axis=0)
            diag = jnp.sum(jnp.where(ri == ci, gpb, 0.0), axis=0,
                           keepdims=True)
            adj = gpb + diag
            xx = x_ref[bb]
            for w_ref, bias_ref in ((w0_ref, b0_ref), (w1_ref, b1_ref)):
                xp = jnp.concatenate([xx, zpad], axis=0)
                h = jnp.dot(adj, xp, preferred_element_type=jnp.float32)
                hw = jnp.dot(h, w_ref[...], preferred_element_type=jnp.float32)
                xx = jnp.maximum(hw + bias_ref[...], 0.0) + xx
            out_ref[bb, :, :] = jnp.mean(xx, axis=0, keepdims=True)

    gp_specs = [
        pl.BlockSpec((1, rpw, LP), lambda i, s=s: (nslab * i + s, 0, 0))
        for s in range(nslab)
    ]
    return pl.pallas_call(
        body,
        grid=(nb // bpg,),
        in_specs=gp_specs + [
            pl.BlockSpec((bpg, L, D), lambda i: (i, 0, 0)),
            pl.BlockSpec((D, D), lambda i: (0, 0)),
            pl.BlockSpec((1, D), lambda i: (0, 0)),
            pl.BlockSpec((D, D), lambda i: (0, 0)),
            pl.BlockSpec((1, D), lambda i: (0, 0)),
        ],
        out_specs=pl.BlockSpec((bpg, 1, D), lambda i: (i, 0, 0)),
        out_shape=jax.ShapeDtypeStruct((nb, 1, D), jnp.float32),
    )(*([gp] * nslab), x, w0, b0, w1, b1).reshape(nb, D)


@jax.jit
def kernel(event_type, enc_output, slf_attn_mask, non_pad_mask, ui_adj,
           W0, b0, W1, b1):
    ev = (event_type - 1).astype(jnp.int32).reshape(B * L)
    a_tail = jnp.pad(ui_adj[:, AMAIN:], ((0, 0), (0, 128 - (NUM_TYPES - AMAIN))))
    eye = jnp.eye(W0.shape[0], dtype=jnp.float32)
    wbd0 = (eye[:, None, :, None] * W0[:, :, None, :]).reshape(D, D)
    wbd1 = (eye[:, None, :, None] * W1[:, :, None, :]).reshape(D, D)
    b0r = b0.reshape(1, D)
    b1r = b1.reshape(1, D)

    nb = B // NSPLIT
    outs = []
    for s in range(NSPLIT):
        gp = _sc_gather_g(ev[s * nb * L:(s + 1) * nb * L], ui_adj, a_tail, nb)
        outs.append(_tc_encoder(
            gp, enc_output[s * nb:(s + 1) * nb], wbd0, b0r, wbd1, b1r, nb))
    return jnp.concatenate(outs, axis=0)


# speedup vs baseline: 1.0586x; 1.0028x over previous
import functools

import jax
import jax.numpy as jnp
from jax import lax
from jax.experimental import pallas as pl
from jax.experimental.pallas import tpu as pltpu
from jax.experimental.pallas import tpu_sc as plsc

NUM_TYPES = 10000
D = 128
B = 16
L = 200
LP = 256
NW = 32
NSPLIT = 1

AMAIN = 9984


def _sc_gather_g(ev_flat, a, a_tail, nb):
    nwpb = NW // nb
    rpw = L // nwpb
    kr = rpw // 25
    gpr = 16 // kr
    ng = rpw // kr
    nt = ((rpw + 15) // 16) * 16
    mesh = plsc.VectorSubcoreMesh(core_axis_name="c", subcore_axis_name="s")

    @functools.partial(
        pl.kernel,
        mesh=mesh,
        out_type=jax.ShapeDtypeStruct((NW, rpw, LP), jnp.float32),
        scratch_types=[
            pltpu.VMEM((LP,), jnp.int32),
            pltpu.VMEM((nt,), jnp.int32),
            pltpu.VMEM((nt // 16, 16), jnp.int32),
            pltpu.VMEM((nt, 128), jnp.float32),
            pltpu.VMEM((kr // 2, AMAIN), jnp.float32),
            pltpu.VMEM((kr // 2, AMAIN), jnp.float32),
            pltpu.VMEM((kr // 2, AMAIN), jnp.float32),
            pltpu.VMEM((kr // 2, AMAIN), jnp.float32),
            pltpu.VMEM((rpw, LP), jnp.float32),
            pltpu.SemaphoreType.DMA,
            pltpu.SemaphoreType.DMA,
            pltpu.SemaphoreType.DMA,
        ],
        compiler_params=pltpu.CompilerParams(needs_layout_passes=False),
    )
    def k(ev_hbm, a_hbm, atail_hbm, out_hbm,
          ev_v, rids_v, rids2_v, tail_v, buf0a, buf0b, buf1a, buf1b, rows_v,
          sem_t, s0, s1):
        bufs = ((buf0a, buf0b), (buf1a, buf1b))
        sems = (s0, s1)
        wid = lax.axis_index("s") * 2 + lax.axis_index("c")
        b = wid // nwpb
        i0 = (wid % nwpb) * rpw

        zeros16 = jnp.zeros((16,), jnp.int32)
        for cc in range(L // 16, LP // 16):
            ev_v[pl.ds(cc * 16, 16)] = zeros16
        pltpu.sync_copy(ev_hbm.at[pl.ds(b * L, L)], ev_v.at[pl.ds(0, L)])

        for cc in range(nt // 16):
            ids = ev_v[pl.ds(i0 + cc * 16, 16)]
            rids_v[pl.ds(cc * 16, 16)] = ids
            rids2_v[cc, pl.ds(0, 16)] = ids
        cp_tail = pltpu.async_copy(atail_hbm.at[rids_v], tail_v, sem_t)

        kh = kr // 2

        def fire(g, t):
            for h in range(2):
                idx = rids2_v.at[g // gpr, pl.ds((g % gpr) * kr + h * kh, kh)]
                pltpu.async_copy(a_hbm.at[idx, pl.ds(0, AMAIN)],
                                 bufs[t][h], sems[t])

        def wait_buf(t):
            for h in range(2):
                pltpu.make_async_copy(
                    a_hbm.at[pl.ds(0, kh), pl.ds(0, AMAIN)],
                    bufs[t][h], sems[t]).wait()

        fire(0, 0)
        cp_tail.wait()

        def extract_group(g, bufpair):
            for rr in range(kr):
                r = g * kr + rr
                frr = jnp.full((16,), rr % kh, jnp.int32)
                fr = jnp.full((16,), r, jnp.int32)
                for cc in range(LP // 16):
                    ev_c = ev_v[pl.ds(cc * 16, 16)]
                    vm = plsc.load_gather(
                        bufpair[rr // kh], [frr, jnp.minimum(ev_c, AMAIN - 1)])
                    vt = plsc.load_gather(
                        tail_v, [fr, jnp.maximum(ev_c - AMAIN, 0)])
                    rows_v[r, pl.ds(cc * 16, 16)] = jnp.where(
                        ev_c < AMAIN, vm, vt)

        def step(q, carry):
            for par in range(2):
                g = q * 2 + par
                fire(g + 1, 1 - par)
                wait_buf(par)
                extract_group(g, bufs[par])
            return carry

        lax.fori_loop(0, ng // 2, step, 0)
        wait_buf(0)
        extract_group(ng - 1, bufs[0])

        pltpu.sync_copy(rows_v, out_hbm.at[wid])

    return k(ev_flat, a, a_tail)


def _tc_encoder(gp, x, w0, b0, w1, b1, nb):
    nwpb = NW // nb
    rpw = L // nwpb
    bpg = 8
    nslab = nwpb * bpg

    def body(*refs):
        gp_refs = refs[:nslab]
        x_ref, w0_ref, b0_ref, w1_ref, b1_ref, out_ref = refs[nslab:]
        ri = lax.broadcasted_iota(jnp.int32, (L, LP), 0)
        ci = lax.broadcasted_iota(jnp.int32, (L, LP), 1)
        zpad = jnp.zeros((LP - L, D), jnp.float32)
        for bb in range(bpg):
            gpb = jnp.concatenate(
                [gr[0] for gr in gp_refs[bb * nwpb:(bb + 1) * nwpb]], axis=0)
            diag = jnp.sum(jnp.where(ri == ci, gpb, 0.0), axis=0,
                           keepdims=True)
            adj = gpb + diag
            xx = x_ref[bb]
            for w_ref, bias_ref in ((w0_ref, b0_ref), (w1_ref, b1_ref)):
                xp = jnp.concatenate([xx, zpad], axis=0)
                h = jnp.dot(adj, xp, preferred_element_type=jnp.float32)
                hw = jnp.dot(h, w_ref[...], preferred_element_type=jnp.float32)
                xx = jnp.maximum(hw + bias_ref[...], 0.0) + xx
            out_ref[bb, :, :] = jnp.mean(xx, axis=0, keepdims=True)

    gp_specs = [
        pl.BlockSpec((1, rpw, LP), lambda i, s=s: (nslab * i + s, 0, 0))
        for s in range(nslab)
    ]
    return pl.pallas_call(
        body,
        grid=(nb // bpg,),
        in_specs=gp_specs + [
            pl.BlockSpec((bpg, L, D), lambda i: (i, 0, 0)),
            pl.BlockSpec((D, D), lambda i: (0, 0)),
            pl.BlockSpec((1, D), lambda i: (0, 0)),
            pl.BlockSpec((D, D), lambda i: (0, 0)),
            pl.BlockSpec((1, D), lambda i: (0, 0)),
        ],
        out_specs=pl.BlockSpec((bpg, 1, D), lambda i: (i, 0, 0)),
        out_shape=jax.ShapeDtypeStruct((nb, 1, D), jnp.float32),
    )(*([gp] * nslab), x, w0, b0, w1, b1).reshape(nb, D)


@jax.jit
def kernel(event_type, enc_output, slf_attn_mask, non_pad_mask, ui_adj,
           W0, b0, W1, b1):
    ev = (event_type - 1).astype(jnp.int32).reshape(B * L)
    a_tail = jnp.pad(ui_adj[:, AMAIN:], ((0, 0), (0, 128 - (NUM_TYPES - AMAIN))))
    eye = jnp.eye(W0.shape[0], dtype=jnp.float32)
    wbd0 = (eye[:, None, :, None] * W0[:, :, None, :]).reshape(D, D)
    wbd1 = (eye[:, None, :, None] * W1[:, :, None, :]).reshape(D, D)
    b0r = b0.reshape(1, D)
    b1r = b1.reshape(1, D)

    nb = B // NSPLIT
    outs = []
    for s in range(NSPLIT):
        gp = _sc_gather_g(ev[s * nb * L:(s + 1) * nb * L], ui_adj, a_tail, nb)
        outs.append(_tc_encoder(
            gp, enc_output[s * nb:(s + 1) * nb], wbd0, b0r, wbd1, b1r, nb))
    return jnp.concatenate(outs, axis=0)
